# baseline (speedup 1.0000x reference)
import functools

import jax
import jax.numpy as jnp
from jax import lax
from jax.experimental import pallas as pl
from jax.experimental.pallas import tpu as pltpu
from jax.experimental.pallas import tpu_sc as plsc

_VOCAB = 1000000
_MOD = _VOCAB - 1
_B, _S = 4, 4096
_N = _B * _S
_D = 128
_M = 1024

_NC, _NS = 2, 16
_NW = _NC * _NS
_C = _N // _NW
_CH = 128


def _mod999999(x):
    m = jnp.int32(_MOD)
    k = jnp.int32(48577)
    msk = jnp.int32(0xFFFFF)
    for _ in range(3):
        x = (x >> 20) * k + (x & msk)
    x = jnp.where(x >= m, x - m, x)
    x = jnp.where(x < 0, x + m, x)
    return x


def _sc_body(tok_hbm, table_hbm, h_hbm,
             tok_v, idx_bi_v, idx_tri_v, rows_bi_v, rows_tri_v,
             sem_bi, sem_tri):
    c = lax.axis_index("c")
    s = lax.axis_index("s")
    wid = s * _NC + c
    base = wid * _C

    pltpu.sync_copy(tok_hbm.at[pl.ds(base, _C)], tok_v.at[pl.ds(8, _C)])

    @pl.when(base > 0)
    def _():
        pltpu.sync_copy(tok_hbm.at[pl.ds(base - 8, 8)], tok_v.at[pl.ds(0, 8)])

    for ch in range(_C // _CH):
        cb = ch * _CH
        for j in range(_CH // 16):
            off = cb + j * 16
            t0 = tok_v[pl.ds(8 + off, 16)]
            tm1 = tok_v[pl.ds(7 + off, 16)]
            tm2 = tok_v[pl.ds(6 + off, 16)]
            a = t0 * jnp.int32(36313)
            b = tm1 * jnp.int32(27191)
            g = tm2 * jnp.int32(51497)
            hb = _mod999999(a ^ b)
            ht = _mod999999(a ^ b ^ g)
            col = (base + off + lax.iota(jnp.int32, 16)) & jnp.int32(_S - 1)
            hb = jnp.where(col == 0, jnp.int32(_MOD), hb)
            ht = jnp.where(col <= 1, jnp.int32(_MOD), ht)
            idx_bi_v[pl.ds(j * 16, 16)] = hb
            idx_tri_v[pl.ds(j * 16, 16)] = ht

        cp_bi = pltpu.async_copy(table_hbm.at[idx_bi_v], rows_bi_v, sem_bi)
        cp_tri = pltpu.async_copy(table_hbm.at[idx_tri_v], rows_tri_v, sem_tri)
        cp_bi.wait()
        cp_tri.wait()

        def _addrow(r, carry):
            for v in range(_D // 16):
                sl = pl.ds(v * 16, 16)
                rows_bi_v[r, sl] = rows_bi_v[r, sl] + rows_tri_v[r, sl]
            return carry

        lax.fori_loop(0, _CH, _addrow, 0)
        pltpu.sync_copy(rows_bi_v, h_hbm.at[pl.ds(base + cb, _CH)])


_sc_gather = functools.partial(
    pl.kernel,
    mesh=plsc.VectorSubcoreMesh(core_axis_name="c", subcore_axis_name="s"),
    out_type=jax.ShapeDtypeStruct((_N, _D), jnp.float32),
    scratch_types=[
        pltpu.VMEM((_C + 8,), jnp.int32),
        pltpu.VMEM((_CH,), jnp.int32),
        pltpu.VMEM((_CH,), jnp.int32),
        pltpu.VMEM((_CH, _D), jnp.float32),
        pltpu.VMEM((_CH, _D), jnp.float32),
        pltpu.SemaphoreType.DMA,
        pltpu.SemaphoreType.DMA,
    ],
)(_sc_body)


def _mm_body(scale_ref, h_ref, w_ref, o_ref):
    acc = lax.dot_general(h_ref[...], w_ref[...],
                          (((1,), (1,)), ((), ())),
                          preferred_element_type=jnp.float32)
    o_ref[...] = acc * scale_ref[0]


def _matmul(h, w, scale):
    bm = 512
    return pl.pallas_call(
        _mm_body,
        grid=(_N // bm,),
        in_specs=[
            pl.BlockSpec(memory_space=pltpu.SMEM),
            pl.BlockSpec((bm, _D), lambda i: (i, 0)),
            pl.BlockSpec((_M, _D), lambda i: (0, 0)),
        ],
        out_specs=pl.BlockSpec((bm, _M), lambda i: (i, 0)),
        out_shape=jax.ShapeDtypeStruct((_N, _M), jnp.float32),
    )(scale.reshape(1), h, w)


def kernel(token_ids, embed_table, proj_w, scale):
    tok = token_ids.reshape(_N)
    h = _sc_gather(tok, embed_table)
    out = _matmul(h, proj_w, scale.astype(jnp.float32))
    return out.reshape(_B, _S, _M)


# speedup vs baseline: 1.0662x; 1.0662x over previous
import functools

import jax
import jax.numpy as jnp
from jax import lax
from jax.experimental import pallas as pl
from jax.experimental.pallas import tpu as pltpu
from jax.experimental.pallas import tpu_sc as plsc

_VOCAB = 1000000
_MOD = _VOCAB - 1
_B, _S = 4, 4096
_N = _B * _S
_D = 128
_M = 1024

_NC, _NS = 2, 16
_NW = _NC * _NS
_CH = _S // _NW


def _mod999999(x):
    m = jnp.int32(_MOD)
    k = jnp.int32(48577)
    msk = jnp.int32(0xFFFFF)
    for _ in range(3):
        x = (x >> 20) * k + (x & msk)
    x = jnp.where(x >= m, x - m, x)
    x = jnp.where(x < 0, x + m, x)
    return x


def _sc_body(slab, tok_hbm, table_hbm, h_hbm,
             tok_v, idx_bi_v, idx_tri_v, rows_bi_v, rows_tri_v,
             sem_bi, sem_tri):
    c = lax.axis_index("c")
    s = lax.axis_index("s")
    wid = s * _NC + c
    base = slab * _S + wid * _CH

    pltpu.sync_copy(tok_hbm.at[pl.ds(base, _CH)], tok_v.at[pl.ds(8, _CH)])

    @pl.when(base > 0)
    def _():
        pltpu.sync_copy(tok_hbm.at[pl.ds(base - 8, 8)], tok_v.at[pl.ds(0, 8)])

    for j in range(_CH // 16):
        off = j * 16
        t0 = tok_v[pl.ds(8 + off, 16)]
        tm1 = tok_v[pl.ds(7 + off, 16)]
        tm2 = tok_v[pl.ds(6 + off, 16)]
        a = t0 * jnp.int32(36313)
        b = tm1 * jnp.int32(27191)
        g = tm2 * jnp.int32(51497)
        hb = _mod999999(a ^ b)
        ht = _mod999999(a ^ b ^ g)
        col = (base + off + lax.iota(jnp.int32, 16)) & jnp.int32(_S - 1)
        hb = jnp.where(col == 0, jnp.int32(_MOD), hb)
        ht = jnp.where(col <= 1, jnp.int32(_MOD), ht)
        idx_bi_v[pl.ds(off, 16)] = hb
        idx_tri_v[pl.ds(off, 16)] = ht

    cp_bi = pltpu.async_copy(table_hbm.at[idx_bi_v], rows_bi_v, sem_bi)
    cp_tri = pltpu.async_copy(table_hbm.at[idx_tri_v], rows_tri_v, sem_tri)
    cp_bi.wait()
    cp_tri.wait()

    def _addrow(r, carry):
        for v in range(_D // 16):
            sl = pl.ds(v * 16, 16)
            rows_bi_v[r, sl] = rows_bi_v[r, sl] + rows_tri_v[r, sl]
        return carry

    lax.fori_loop(0, _CH, _addrow, 0)
    pltpu.sync_copy(rows_bi_v, h_hbm.at[pl.ds(wid * _CH, _CH)])


def _make_sc(slab):
    return pl.kernel(
        functools.partial(_sc_body, slab),
        mesh=plsc.VectorSubcoreMesh(core_axis_name="c", subcore_axis_name="s"),
        out_type=jax.ShapeDtypeStruct((_S, _D), jnp.float32),
        scratch_types=[
            pltpu.VMEM((_CH + 8,), jnp.int32),
            pltpu.VMEM((_CH,), jnp.int32),
            pltpu.VMEM((_CH,), jnp.int32),
            pltpu.VMEM((_CH, _D), jnp.float32),
            pltpu.VMEM((_CH, _D), jnp.float32),
            pltpu.SemaphoreType.DMA,
            pltpu.SemaphoreType.DMA,
        ],
    )


_sc_gathers = [_make_sc(k) for k in range(_B)]

_BM = 512
_SLAB_BLOCKS = _S // _BM


def _mm_first_body(scale_ref, h_ref, w_ref, o_ref):
    acc = lax.dot_general(h_ref[...], w_ref[...],
                          (((1,), (1,)), ((), ())),
                          preferred_element_type=jnp.float32)
    o_ref[...] = acc * scale_ref[0]


def _mm_chain_body(ob_ref, scale_ref, h_ref, w_ref, o_ref):
    del ob_ref
    _mm_first_body(scale_ref, h_ref, w_ref, o_ref)


def _matmul_slab(k, out_buf, h, w, scale):
    if k == 0:
        return pl.pallas_call(
            _mm_first_body,
            grid=(_SLAB_BLOCKS,),
            in_specs=[
                pl.BlockSpec(memory_space=pltpu.SMEM),
                pl.BlockSpec((_BM, _D), lambda i: (i, 0)),
                pl.BlockSpec((_M, _D), lambda i: (0, 0)),
            ],
            out_specs=pl.BlockSpec((_BM, _M), lambda i: (i, 0)),
            out_shape=jax.ShapeDtypeStruct((_N, _M), jnp.float32),
        )(scale, h, w)
    return pl.pallas_call(
        _mm_chain_body,
        grid=(_SLAB_BLOCKS,),
        in_specs=[
            pl.BlockSpec(memory_space=pl.ANY),
            pl.BlockSpec(memory_space=pltpu.SMEM),
            pl.BlockSpec((_BM, _D), lambda i: (i, 0)),
            pl.BlockSpec((_M, _D), lambda i: (0, 0)),
        ],
        out_specs=pl.BlockSpec((_BM, _M), lambda i, k=k: (i + k * _SLAB_BLOCKS, 0)),
        out_shape=jax.ShapeDtypeStruct((_N, _M), jnp.float32),
        input_output_aliases={0: 0},
    )(out_buf, scale, h, w)


def kernel(token_ids, embed_table, proj_w, scale):
    tok = token_ids.reshape(_N)
    scale1 = scale.astype(jnp.float32).reshape(1)
    hs = [_sc_gathers[k](tok, embed_table) for k in range(_B)]
    out = None
    for k in range(_B):
        out = _matmul_slab(k, out, hs[k], proj_w, scale1)
    return out.reshape(_B, _S, _M)
